# baseline (speedup 1.0000x reference)
import functools

import jax
import jax.numpy as jnp
from jax import lax
from jax.experimental import pallas as pl
from jax.experimental.pallas import tpu as pltpu
from jax.experimental.pallas import tpu_sc as plsc

B, L, V, D, H = 4096, 200, 1000000, 64, 64
N = B * L

NC, NS = 2, 16
NW = NC * NS
ROWS_PER_W = N // NW
CHUNK = 128
NCHUNK = ROWS_PER_W // CHUNK

ROW_BLK = 8192
NBLK = (V + ROW_BLK - 1) // ROW_BLK
VP = NBLK * ROW_BLK
HALF = ROW_BLK // 2


def _proj_body(embt_ref, w_ref, out_ref):
    y = lax.dot_general(
        embt_ref[...], w_ref[...],
        dimension_numbers=(((0,), (1,)), ((), ())),
        preferred_element_type=jnp.float32,
    ).astype(jnp.bfloat16)
    out_ref[:, :H] = y[:HALF]
    out_ref[:, H:] = y[HALF:]


def _project(embt, W):
    return pl.pallas_call(
        _proj_body,
        grid=(NBLK,),
        in_specs=[
            pl.BlockSpec((D, ROW_BLK), lambda i: (0, i)),
            pl.BlockSpec((H, D), lambda i: (0, 0)),
        ],
        out_specs=pl.BlockSpec((HALF, 2 * H), lambda i: (i, 0)),
        out_shape=jax.ShapeDtypeStruct((VP // 2, 2 * H), jnp.bfloat16),
    )(embt, W)


_mesh = plsc.VectorSubcoreMesh(core_axis_name="c", subcore_axis_name="s")


NBUF = 8
DEPTH = NBUF // 2
NLAP = NCHUNK // NBUF
SLAB = 8
BHALF = B // 2


def _sig16(v):
    return ((v >> 13) << 13) + ((v & (HALF - 1)) << 1) + ((v >> 12) & 1)


@functools.partial(
    pl.kernel,
    mesh=_mesh,
    compiler_params=pltpu.CompilerParams(
        use_tc_tiling_on_sc=False, needs_layout_passes=False),
    out_type=jax.ShapeDtypeStruct((N, H), jnp.bfloat16),
    scratch_types=[
        pltpu.VMEM((SLAB * B,), jnp.int32),
        pltpu.VMEM((NBUF, CHUNK), jnp.int32),
        pltpu.VMEM((NBUF, CHUNK, H), jnp.bfloat16),
        [pltpu.SemaphoreType.DMA] * NBUF,
        [pltpu.SemaphoreType.DMA] * NBUF,
    ],
)
def _gather(proj_hbm, sentt_hbm, out_hbm, slab_v, cidx_v, rows_v, gsem, ssem):
    wid = lax.axis_index("s") * NC + lax.axis_index("c")
    base = wid * ROWS_PER_W
    l_lo = jnp.minimum(base // B, L - SLAB)

    for r in range(SLAB):
        pltpu.sync_copy(sentt_hbm.at[l_lo + r],
                        slab_v.at[pl.ds(r * B, B)])

    iota = jax.lax.iota(jnp.int32, 16)

    def build(j, b):
        m0 = base + j * CHUNK
        l = m0 // B
        col0 = (m0 - l * B) >> 1
        flat0 = (l - l_lo) * B + col0
        dst = cidx_v.at[b]
        for t in range(4):
            ev = slab_v[pl.ds(flat0 + 16 * t, 16)]
            od = slab_v[pl.ds(flat0 + BHALF + 16 * t, 16)]
            plsc.store_scatter(dst, [iota * 2 + 32 * t], _sig16(ev))
            plsc.store_scatter(dst, [iota * 2 + 32 * t + 1], _sig16(od))

    def g_copy(b):
        return pltpu.make_async_copy(
            proj_hbm.at[cidx_v.at[b]], rows_v.at[b], gsem[b])

    def s_copy(j, b):
        return pltpu.make_async_copy(
            rows_v.at[b], out_hbm.at[pl.ds(base + j * CHUNK, CHUNK)],
            ssem[b])

    def emit(j, b, issue_next, wait_next_store):
        g_copy(b).wait()
        s_copy(j, b).start()
        if issue_next:
            nxt = j + DEPTH
            b2 = (b + DEPTH) % NBUF
            if wait_next_store:
                s_copy(nxt - NBUF, b2).wait()
            build(nxt, b2)
            g_copy(b2).start()

    for b in range(DEPTH):
        build(b, b)
        g_copy(b).start()
    for b in range(NBUF):
        emit(b, b, True, b >= DEPTH)

    def lap(t, carry):
        j0 = t * NBUF
        for b in range(NBUF):
            emit(j0 + b, b, True, True)
        return carry

    lax.fori_loop(1, NLAP - 1, lap, 0)

    j0 = (NLAP - 1) * NBUF
    for b in range(NBUF):
        emit(j0 + b, b, b < DEPTH, True)
    for b in range(NBUF):
        s_copy(j0 + b, b).wait()


def _trans_body(i_ref, g_ref, out_ref):
    blk = g_ref[...]
    dn = (((0,), (1,)), ((), ()))
    out_ref[0, :, :BHALF] = lax.dot_general(
        i_ref[...], blk[:, :H], dn, preferred_element_type=jnp.float32)
    out_ref[0, :, BHALF:] = lax.dot_general(
        i_ref[...], blk[:, H:], dn, preferred_element_type=jnp.float32)


def _transpose_out(g2, eye):
    return pl.pallas_call(
        _trans_body,
        grid=(L,),
        in_specs=[
            pl.BlockSpec((H, H), lambda i: (0, 0)),
            pl.BlockSpec((BHALF, 2 * H), lambda i: (i, 0)),
        ],
        out_specs=pl.BlockSpec((1, H, B), lambda i: (i, 0, 0)),
        out_shape=jax.ShapeDtypeStruct((L, H, B), jnp.float32),
    )(eye, g2)


def kernel(sent1, sent2, emb, W):
    proj = _project(emb.T, W).reshape(VP, H)
    eye = jnp.eye(H, dtype=jnp.bfloat16)
    g1 = _gather(proj, sent1.T.astype(jnp.int32))
    g2 = _gather(proj, sent2.T.astype(jnp.int32))
    o1 = _transpose_out(g1.reshape(N // 2, 2 * H), eye)
    o2 = _transpose_out(g2.reshape(N // 2, 2 * H), eye)
    return (o1.transpose(2, 0, 1), o2.transpose(2, 0, 1))


# speedup vs baseline: 2.3121x; 2.3121x over previous
import functools

import jax
import jax.numpy as jnp
from jax import lax
from jax.experimental import pallas as pl
from jax.experimental.pallas import tpu as pltpu
from jax.experimental.pallas import tpu_sc as plsc

B, L, V, D, H = 4096, 200, 1000000, 64, 64
N = B * L

NC, NS = 2, 16
NW = NC * NS
ROWS_PER_W = N // NW
CHUNK = 128
NCHUNK = ROWS_PER_W // CHUNK

ROW_BLK = 8192
NBLK = (V + ROW_BLK - 1) // ROW_BLK
VP = NBLK * ROW_BLK
HALF = ROW_BLK // 2


def _proj_body(embt_ref, w_ref, out_ref):
    y = lax.dot_general(
        embt_ref[...], w_ref[...],
        dimension_numbers=(((0,), (1,)), ((), ())),
        preferred_element_type=jnp.float32,
    )
    out_ref[:, :H] = y[:HALF]
    out_ref[:, H:] = y[HALF:]


def _project(embt, W):
    return pl.pallas_call(
        _proj_body,
        grid=(NBLK,),
        in_specs=[
            pl.BlockSpec((D, ROW_BLK), lambda i: (0, i)),
            pl.BlockSpec((H, D), lambda i: (0, 0)),
        ],
        out_specs=pl.BlockSpec((HALF, 2 * H), lambda i: (i, 0)),
        out_shape=jax.ShapeDtypeStruct((VP // 2, 2 * H), jnp.float32),
    )(embt, W)


_mesh = plsc.VectorSubcoreMesh(core_axis_name="c", subcore_axis_name="s")


NBUF = 8
DEPTH = NBUF // 2
NLAP = NCHUNK // NBUF
SLAB = 8
BHALF = B // 2


def _sig16(v):
    return ((v >> 13) << 13) + ((v & (HALF - 1)) << 1) + ((v >> 12) & 1)


@functools.partial(
    pl.kernel,
    mesh=_mesh,
    compiler_params=pltpu.CompilerParams(
        use_tc_tiling_on_sc=False, needs_layout_passes=False),
    out_type=jax.ShapeDtypeStruct((N, H), jnp.float32),
    scratch_types=[
        pltpu.VMEM((SLAB * B,), jnp.int32),
        pltpu.VMEM((NBUF, CHUNK), jnp.int32),
        pltpu.VMEM((NBUF, CHUNK, H), jnp.float32),
        [pltpu.SemaphoreType.DMA] * NBUF,
        [pltpu.SemaphoreType.DMA] * NBUF,
    ],
)
def _gather(proj_hbm, sentt_hbm, out_hbm, slab_v, cidx_v, rows_v, gsem, ssem):
    wid = lax.axis_index("s") * NC + lax.axis_index("c")
    base = wid * ROWS_PER_W
    l_lo = jnp.minimum(base // B, L - SLAB)

    for r in range(SLAB):
        pltpu.sync_copy(sentt_hbm.at[l_lo + r],
                        slab_v.at[pl.ds(r * B, B)])

    iota = jax.lax.iota(jnp.int32, 16)

    def build(j, b):
        m0 = base + j * CHUNK
        l = m0 // B
        col0 = (m0 - l * B) >> 1
        flat0 = (l - l_lo) * B + col0
        dst = cidx_v.at[b]
        for t in range(4):
            ev = slab_v[pl.ds(flat0 + 16 * t, 16)]
            od = slab_v[pl.ds(flat0 + BHALF + 16 * t, 16)]
            plsc.store_scatter(dst, [iota * 2 + 32 * t], _sig16(ev))
            plsc.store_scatter(dst, [iota * 2 + 32 * t + 1], _sig16(od))

    def g_copy(b):
        return pltpu.make_async_copy(
            proj_hbm.at[cidx_v.at[b]], rows_v.at[b], gsem[b])

    def s_copy(j, b):
        return pltpu.make_async_copy(
            rows_v.at[b], out_hbm.at[pl.ds(base + j * CHUNK, CHUNK)],
            ssem[b])

    def emit(j, b, issue_next, wait_next_store):
        g_copy(b).wait()
        s_copy(j, b).start()
        if issue_next:
            nxt = j + DEPTH
            b2 = (b + DEPTH) % NBUF
            if wait_next_store:
                s_copy(nxt - NBUF, b2).wait()
            build(nxt, b2)
            g_copy(b2).start()

    for b in range(DEPTH):
        build(b, b)
        g_copy(b).start()
    for b in range(NBUF):
        emit(b, b, True, b >= DEPTH)

    def lap(t, carry):
        j0 = t * NBUF
        for b in range(NBUF):
            emit(j0 + b, b, True, True)
        return carry

    lax.fori_loop(1, NLAP - 1, lap, 0)

    j0 = (NLAP - 1) * NBUF
    for b in range(NBUF):
        emit(j0 + b, b, b < DEPTH, True)
    for b in range(NBUF):
        s_copy(j0 + b, b).wait()


LSTEP = 2


def _trans_body(i_ref, g_ref, out_ref):
    dn = (((0,), (1,)), ((), ()))
    for s in range(LSTEP):
        blk = g_ref[pl.ds(s * BHALF, BHALF), :]
        out_ref[s, :, :BHALF] = lax.dot_general(
            i_ref[...], blk[:, :H], dn, preferred_element_type=jnp.float32)
        out_ref[s, :, BHALF:] = lax.dot_general(
            i_ref[...], blk[:, H:], dn, preferred_element_type=jnp.float32)


def _transpose_out(g2, eye):
    return pl.pallas_call(
        _trans_body,
        grid=(L // LSTEP,),
        in_specs=[
            pl.BlockSpec((H, H), lambda i: (0, 0)),
            pl.BlockSpec((LSTEP * BHALF, 2 * H), lambda i: (i, 0)),
        ],
        out_specs=pl.BlockSpec((LSTEP, H, B), lambda i: (i, 0, 0)),
        out_shape=jax.ShapeDtypeStruct((L, H, B), jnp.float32),
    )(eye, g2)


def kernel(sent1, sent2, emb, W):
    proj = _project(emb.T, W).reshape(VP, H)
    eye = jnp.eye(H, dtype=jnp.float32)
    g1 = _gather(proj, sent1.T.astype(jnp.int32))
    g2 = _gather(proj, sent2.T.astype(jnp.int32))
    o1 = _transpose_out(g1.reshape(N // 2, 2 * H), eye)
    o2 = _transpose_out(g2.reshape(N // 2, 2 * H), eye)
    return (o1.transpose(2, 0, 1), o2.transpose(2, 0, 1))


# speedup vs baseline: 2.4150x; 1.0445x over previous
import functools

import jax
import jax.numpy as jnp
from jax import lax
from jax.experimental import pallas as pl
from jax.experimental.pallas import tpu as pltpu
from jax.experimental.pallas import tpu_sc as plsc

B, L, V, D, H = 4096, 200, 1000000, 64, 64
N = B * L

NC, NS = 2, 16
NW = NC * NS
ROWS_PER_W = N // NW
CHUNK = 128
NCHUNK = ROWS_PER_W // CHUNK

ROW_BLK = 8192
NBLK = (V + ROW_BLK - 1) // ROW_BLK
VP = NBLK * ROW_BLK
HALF = ROW_BLK // 2


def _proj_body(embt_ref, w_ref, out_ref):
    y = lax.dot_general(
        embt_ref[...], w_ref[...],
        dimension_numbers=(((0,), (1,)), ((), ())),
        preferred_element_type=jnp.float32,
    )
    out_ref[:, :H] = y[:HALF]
    out_ref[:, H:] = y[HALF:]


def _project(embt, W):
    return pl.pallas_call(
        _proj_body,
        grid=(NBLK,),
        in_specs=[
            pl.BlockSpec((D, ROW_BLK), lambda i: (0, i)),
            pl.BlockSpec((H, D), lambda i: (0, 0)),
        ],
        out_specs=pl.BlockSpec((HALF, 2 * H), lambda i: (i, 0)),
        out_shape=jax.ShapeDtypeStruct((VP // 2, 2 * H), jnp.float32),
    )(embt, W)


_mesh = plsc.VectorSubcoreMesh(core_axis_name="c", subcore_axis_name="s")


NBUF = 8
DEPTH = NBUF // 2
NLAP = NCHUNK // NBUF
SLAB = 8
BHALF = B // 2


def _sig16(v):
    return ((v >> 13) << 13) + ((v & (HALF - 1)) << 1) + ((v >> 12) & 1)


@functools.partial(
    pl.kernel,
    mesh=_mesh,
    compiler_params=pltpu.CompilerParams(
        use_tc_tiling_on_sc=False, needs_layout_passes=False),
    out_type=jax.ShapeDtypeStruct((N, H), jnp.float32),
    scratch_types=[
        pltpu.VMEM((SLAB * B,), jnp.int32),
        pltpu.VMEM((NBUF, CHUNK), jnp.int32),
        pltpu.VMEM((NBUF, CHUNK, H), jnp.float32),
        [pltpu.SemaphoreType.DMA] * NBUF,
        [pltpu.SemaphoreType.DMA] * NBUF,
    ],
)
def _gather(proj_hbm, sentt_hbm, out_hbm, slab_v, cidx_v, rows_v, gsem, ssem):
    wid = lax.axis_index("s") * NC + lax.axis_index("c")
    base = wid * ROWS_PER_W
    l_lo = jnp.minimum(base // B, L - SLAB)

    for r in range(SLAB):
        pltpu.sync_copy(sentt_hbm.at[l_lo + r],
                        slab_v.at[pl.ds(r * B, B)])

    iota = jax.lax.iota(jnp.int32, 16)

    def build(j, b):
        m0 = base + j * CHUNK
        l = m0 // B
        col0 = (m0 - l * B) >> 1
        flat0 = (l - l_lo) * B + col0
        dst = cidx_v.at[b]
        for t in range(4):
            ev = slab_v[pl.ds(flat0 + 16 * t, 16)]
            od = slab_v[pl.ds(flat0 + BHALF + 16 * t, 16)]
            plsc.store_scatter(dst, [iota * 2 + 32 * t], _sig16(ev))
            plsc.store_scatter(dst, [iota * 2 + 32 * t + 1], _sig16(od))

    def g_copy(b):
        return pltpu.make_async_copy(
            proj_hbm.at[cidx_v.at[b]], rows_v.at[b], gsem[b])

    def s_copy(j, b):
        return pltpu.make_async_copy(
            rows_v.at[b], out_hbm.at[pl.ds(base + j * CHUNK, CHUNK)],
            ssem[b])

    def emit(j, b, issue_next, wait_next_store):
        g_copy(b).wait()
        s_copy(j, b).start()
        if issue_next:
            nxt = j + DEPTH
            b2 = (b + DEPTH) % NBUF
            if wait_next_store:
                s_copy(nxt - NBUF, b2).wait()
            build(nxt, b2)
            g_copy(b2).start()

    for b in range(DEPTH):
        build(b, b)
        g_copy(b).start()
    for b in range(NBUF):
        emit(b, b, True, b >= DEPTH)

    def lap(t, carry):
        j0 = t * NBUF
        for b in range(NBUF):
            emit(j0 + b, b, True, True)
        return carry

    lax.fori_loop(1, NLAP - 1, lap, 0)

    j0 = (NLAP - 1) * NBUF
    for b in range(NBUF):
        emit(j0 + b, b, b < DEPTH, True)
    for b in range(NBUF):
        s_copy(j0 + b, b).wait()


LSTEP = 4


def _trans_body(i_ref, g_ref, out_ref):
    dn = (((0,), (1,)), ((), ()))
    for s in range(LSTEP):
        blk = g_ref[pl.ds(s * BHALF, BHALF), :]
        out_ref[s, :, :BHALF] = lax.dot_general(
            i_ref[...], blk[:, :H], dn, preferred_element_type=jnp.float32)
        out_ref[s, :, BHALF:] = lax.dot_general(
            i_ref[...], blk[:, H:], dn, preferred_element_type=jnp.float32)


def _transpose_out(g2, eye):
    return pl.pallas_call(
        _trans_body,
        grid=(L // LSTEP,),
        in_specs=[
            pl.BlockSpec((H, H), lambda i: (0, 0)),
            pl.BlockSpec((LSTEP * BHALF, 2 * H), lambda i: (i, 0)),
        ],
        out_specs=pl.BlockSpec((LSTEP, H, B), lambda i: (i, 0, 0)),
        out_shape=jax.ShapeDtypeStruct((L, H, B), jnp.float32),
    )(eye, g2)


def kernel(sent1, sent2, emb, W):
    proj = _project(emb.T, W).reshape(VP, H)
    eye = jnp.eye(H, dtype=jnp.float32)
    g1 = _gather(proj, sent1.T.astype(jnp.int32))
    g2 = _gather(proj, sent2.T.astype(jnp.int32))
    o1 = _transpose_out(g1.reshape(N // 2, 2 * H), eye)
    o2 = _transpose_out(g2.reshape(N // 2, 2 * H), eye)
    return (o1.transpose(2, 0, 1), o2.transpose(2, 0, 1))


# speedup vs baseline: 2.4373x; 1.0092x over previous
import functools

import jax
import jax.numpy as jnp
from jax import lax
from jax.experimental import pallas as pl
from jax.experimental.pallas import tpu as pltpu
from jax.experimental.pallas import tpu_sc as plsc

B, L, V, D, H = 4096, 200, 1000000, 64, 64
N = B * L

NC, NS = 2, 16
NW = NC * NS
ROWS_PER_W = N // NW
CHUNK = 128
NCHUNK = ROWS_PER_W // CHUNK

ROW_BLK = 8192
NBLK = (V + ROW_BLK - 1) // ROW_BLK
VP = NBLK * ROW_BLK
HALF = ROW_BLK // 2


def _proj_body(embt_ref, w_ref, out_ref):
    y = lax.dot_general(
        embt_ref[...], w_ref[...],
        dimension_numbers=(((0,), (1,)), ((), ())),
        preferred_element_type=jnp.float32,
    )
    out_ref[:, :H] = y[:HALF]
    out_ref[:, H:] = y[HALF:]


def _project(embt, W):
    return pl.pallas_call(
        _proj_body,
        grid=(NBLK,),
        in_specs=[
            pl.BlockSpec((D, ROW_BLK), lambda i: (0, i)),
            pl.BlockSpec((H, D), lambda i: (0, 0)),
        ],
        out_specs=pl.BlockSpec((HALF, 2 * H), lambda i: (i, 0)),
        out_shape=jax.ShapeDtypeStruct((VP // 2, 2 * H), jnp.float32),
    )(embt, W)


_mesh = plsc.VectorSubcoreMesh(core_axis_name="c", subcore_axis_name="s")


NBUF = 8
DEPTH = NBUF // 2
NLAP = NCHUNK // NBUF
SLAB = 8
BHALF = B // 2


def _sig16(v):
    return ((v >> 13) << 13) + ((v & (HALF - 1)) << 1) + ((v >> 12) & 1)


@functools.partial(
    pl.kernel,
    mesh=_mesh,
    compiler_params=pltpu.CompilerParams(
        use_tc_tiling_on_sc=False, needs_layout_passes=False),
    out_type=jax.ShapeDtypeStruct((N, H), jnp.float32),
    scratch_types=[
        pltpu.VMEM((SLAB * B,), jnp.int32),
        pltpu.VMEM((NBUF, CHUNK), jnp.int32),
        pltpu.VMEM((NBUF, CHUNK, H), jnp.float32),
        [pltpu.SemaphoreType.DMA] * NBUF,
        [pltpu.SemaphoreType.DMA] * NBUF,
    ],
)
def _gather(proj_hbm, sentt_hbm, out_hbm, slab_v, cidx_v, rows_v, gsem, ssem):
    wid = lax.axis_index("s") * NC + lax.axis_index("c")
    base = wid * ROWS_PER_W
    l_lo = jnp.minimum(base // B, L - SLAB)

    for r in range(SLAB):
        pltpu.sync_copy(sentt_hbm.at[l_lo + r],
                        slab_v.at[pl.ds(r * B, B)])

    iota = jax.lax.iota(jnp.int32, 16)

    def build(j, b):
        m0 = base + j * CHUNK
        l = m0 // B
        col0 = (m0 - l * B) >> 1
        flat0 = (l - l_lo) * B + col0
        dst = cidx_v.at[b]
        for t in range(4):
            ev = slab_v[pl.ds(flat0 + 16 * t, 16)]
            od = slab_v[pl.ds(flat0 + BHALF + 16 * t, 16)]
            plsc.store_scatter(dst, [iota * 2 + 32 * t], _sig16(ev))
            plsc.store_scatter(dst, [iota * 2 + 32 * t + 1], _sig16(od))

    def g_copy(b):
        return pltpu.make_async_copy(
            proj_hbm.at[cidx_v.at[b]], rows_v.at[b], gsem[b])

    def s_copy(j, b):
        return pltpu.make_async_copy(
            rows_v.at[b], out_hbm.at[pl.ds(base + j * CHUNK, CHUNK)],
            ssem[b])

    def emit(j, b, issue_next, wait_next_store):
        g_copy(b).wait()
        s_copy(j, b).start()
        if issue_next:
            nxt = j + DEPTH
            b2 = (b + DEPTH) % NBUF
            if wait_next_store:
                s_copy(nxt - NBUF, b2).wait()
            build(nxt, b2)
            g_copy(b2).start()

    for b in range(DEPTH):
        build(b, b)
        g_copy(b).start()
    for b in range(NBUF):
        emit(b, b, True, b >= DEPTH)

    def lap(t, carry):
        j0 = t * NBUF
        for b in range(NBUF):
            emit(j0 + b, b, True, True)
        return carry

    lax.fori_loop(1, NLAP - 1, lap, 0)

    j0 = (NLAP - 1) * NBUF
    for b in range(NBUF):
        emit(j0 + b, b, b < DEPTH, True)
    for b in range(NBUF):
        s_copy(j0 + b, b).wait()


LSTEP = 8


def _trans_body(i_ref, g_ref, out_ref):
    dn = (((0,), (1,)), ((), ()))
    for s in range(LSTEP):
        blk = g_ref[pl.ds(s * BHALF, BHALF), :]
        out_ref[s, :, :BHALF] = lax.dot_general(
            i_ref[...], blk[:, :H], dn, preferred_element_type=jnp.float32)
        out_ref[s, :, BHALF:] = lax.dot_general(
            i_ref[...], blk[:, H:], dn, preferred_element_type=jnp.float32)


def _transpose_out(g2, eye):
    return pl.pallas_call(
        _trans_body,
        grid=(L // LSTEP,),
        in_specs=[
            pl.BlockSpec((H, H), lambda i: (0, 0)),
            pl.BlockSpec((LSTEP * BHALF, 2 * H), lambda i: (i, 0)),
        ],
        out_specs=pl.BlockSpec((LSTEP, H, B), lambda i: (i, 0, 0)),
        out_shape=jax.ShapeDtypeStruct((L, H, B), jnp.float32),
    )(eye, g2)


def kernel(sent1, sent2, emb, W):
    proj = _project(emb.T, W).reshape(VP, H)
    eye = jnp.eye(H, dtype=jnp.float32)
    g1 = _gather(proj, sent1.T.astype(jnp.int32))
    g2 = _gather(proj, sent2.T.astype(jnp.int32))
    o1 = _transpose_out(g1.reshape(N // 2, 2 * H), eye)
    o2 = _transpose_out(g2.reshape(N // 2, 2 * H), eye)
    return (o1.transpose(2, 0, 1), o2.transpose(2, 0, 1))


# speedup vs baseline: 2.5313x; 1.0386x over previous
import functools

import jax
import jax.numpy as jnp
from jax import lax
from jax.experimental import pallas as pl
from jax.experimental.pallas import tpu as pltpu
from jax.experimental.pallas import tpu_sc as plsc

B, L, V, D, H = 4096, 200, 1000000, 64, 64
N = B * L

NC, NS = 2, 16
NW = NC * NS
ROWS_PER_W = N // NW
CHUNK = 128
NCHUNK = ROWS_PER_W // CHUNK

ROW_BLK = 16384
NBLK = (V + ROW_BLK - 1) // ROW_BLK
VP = NBLK * ROW_BLK
HALF = ROW_BLK // 2
SH = ROW_BLK.bit_length() - 1


def _proj_body(embt_ref, w_ref, out_ref):
    y = lax.dot_general(
        embt_ref[...], w_ref[...],
        dimension_numbers=(((0,), (1,)), ((), ())),
        preferred_element_type=jnp.float32,
    )
    out_ref[:, :H] = y[:HALF]
    out_ref[:, H:] = y[HALF:]


def _project(embt, W):
    return pl.pallas_call(
        _proj_body,
        grid=(NBLK,),
        in_specs=[
            pl.BlockSpec((D, ROW_BLK), lambda i: (0, i)),
            pl.BlockSpec((H, D), lambda i: (0, 0)),
        ],
        out_specs=pl.BlockSpec((HALF, 2 * H), lambda i: (i, 0)),
        out_shape=jax.ShapeDtypeStruct((VP // 2, 2 * H), jnp.float32),
    )(embt, W)


_mesh = plsc.VectorSubcoreMesh(core_axis_name="c", subcore_axis_name="s")


NBUF = 8
DEPTH = NBUF // 2
NLAP = NCHUNK // NBUF
SLAB = 8
BHALF = B // 2


def _sig16(v):
    return ((v >> SH) << SH) + ((v & (HALF - 1)) << 1) + ((v >> (SH - 1)) & 1)


@functools.partial(
    pl.kernel,
    mesh=_mesh,
    compiler_params=pltpu.CompilerParams(
        use_tc_tiling_on_sc=False, needs_layout_passes=False),
    out_type=jax.ShapeDtypeStruct((N, H), jnp.float32),
    scratch_types=[
        pltpu.VMEM((SLAB * B,), jnp.int32),
        pltpu.VMEM((NBUF, CHUNK), jnp.int32),
        pltpu.VMEM((NBUF, CHUNK, H), jnp.float32),
        [pltpu.SemaphoreType.DMA] * NBUF,
        [pltpu.SemaphoreType.DMA] * NBUF,
    ],
)
def _gather(proj_hbm, sentt_hbm, out_hbm, slab_v, cidx_v, rows_v, gsem, ssem):
    wid = lax.axis_index("s") * NC + lax.axis_index("c")
    base = wid * ROWS_PER_W
    l_lo = jnp.minimum(base // B, L - SLAB)

    for r in range(SLAB):
        pltpu.sync_copy(sentt_hbm.at[l_lo + r],
                        slab_v.at[pl.ds(r * B, B)])

    iota = jax.lax.iota(jnp.int32, 16)

    def build(j, b):
        m0 = base + j * CHUNK
        l = m0 // B
        col0 = (m0 - l * B) >> 1
        flat0 = (l - l_lo) * B + col0
        dst = cidx_v.at[b]
        for t in range(4):
            ev = slab_v[pl.ds(flat0 + 16 * t, 16)]
            od = slab_v[pl.ds(flat0 + BHALF + 16 * t, 16)]
            plsc.store_scatter(dst, [iota * 2 + 32 * t], _sig16(ev))
            plsc.store_scatter(dst, [iota * 2 + 32 * t + 1], _sig16(od))

    def g_copy(b):
        return pltpu.make_async_copy(
            proj_hbm.at[cidx_v.at[b]], rows_v.at[b], gsem[b])

    def s_copy(j, b):
        return pltpu.make_async_copy(
            rows_v.at[b], out_hbm.at[pl.ds(base + j * CHUNK, CHUNK)],
            ssem[b])

    def emit(j, b, issue_next, wait_next_store):
        g_copy(b).wait()
        s_copy(j, b).start()
        if issue_next:
            nxt = j + DEPTH
            b2 = (b + DEPTH) % NBUF
            if wait_next_store:
                s_copy(nxt - NBUF, b2).wait()
            build(nxt, b2)
            g_copy(b2).start()

    for b in range(DEPTH):
        build(b, b)
        g_copy(b).start()
    for b in range(NBUF):
        emit(b, b, True, b >= DEPTH)

    def lap(t, carry):
        j0 = t * NBUF
        for b in range(NBUF):
            emit(j0 + b, b, True, True)
        return carry

    lax.fori_loop(1, NLAP - 1, lap, 0)

    j0 = (NLAP - 1) * NBUF
    for b in range(NBUF):
        emit(j0 + b, b, b < DEPTH, True)
    for b in range(NBUF):
        s_copy(j0 + b, b).wait()


LSTEP = 8


def _trans_body(i_ref, g_ref, out_ref):
    dn = (((0,), (1,)), ((), ()))
    for s in range(LSTEP):
        blk = g_ref[pl.ds(s * BHALF, BHALF), :]
        out_ref[s, :, :BHALF] = lax.dot_general(
            i_ref[...], blk[:, :H], dn, preferred_element_type=jnp.float32)
        out_ref[s, :, BHALF:] = lax.dot_general(
            i_ref[...], blk[:, H:], dn, preferred_element_type=jnp.float32)


def _transpose_out(g2, eye):
    return pl.pallas_call(
        _trans_body,
        grid=(L // LSTEP,),
        in_specs=[
            pl.BlockSpec((H, H), lambda i: (0, 0)),
            pl.BlockSpec((LSTEP * BHALF, 2 * H), lambda i: (i, 0)),
        ],
        out_specs=pl.BlockSpec((LSTEP, H, B), lambda i: (i, 0, 0)),
        out_shape=jax.ShapeDtypeStruct((L, H, B), jnp.float32),
    )(eye, g2)


def kernel(sent1, sent2, emb, W):
    proj = _project(emb.T, W).reshape(VP, H)
    eye = jnp.eye(H, dtype=jnp.float32)
    g1 = _gather(proj, sent1.T.astype(jnp.int32))
    g2 = _gather(proj, sent2.T.astype(jnp.int32))
    o1 = _transpose_out(g1.reshape(N // 2, 2 * H), eye)
    o2 = _transpose_out(g2.reshape(N // 2, 2 * H), eye)
    return (o1.transpose(2, 0, 1), o2.transpose(2, 0, 1))


# speedup vs baseline: 2.5809x; 1.0196x over previous
import functools

import jax
import jax.numpy as jnp
from jax import lax
from jax.experimental import pallas as pl
from jax.experimental.pallas import tpu as pltpu
from jax.experimental.pallas import tpu_sc as plsc

B, L, V, D, H = 4096, 200, 1000000, 64, 64
N = B * L

NC, NS = 2, 16
NW = NC * NS
ROWS_PER_W = N // NW
CHUNK = 128
NCHUNK = ROWS_PER_W // CHUNK

ROW_BLK = 32768
NBLK = (V + ROW_BLK - 1) // ROW_BLK
VP = NBLK * ROW_BLK
HALF = ROW_BLK // 2
SH = ROW_BLK.bit_length() - 1


def _proj_body(embt_ref, w_ref, out_ref):
    y = lax.dot_general(
        embt_ref[...], w_ref[...],
        dimension_numbers=(((0,), (1,)), ((), ())),
        preferred_element_type=jnp.float32,
    )
    out_ref[:, :H] = y[:HALF]
    out_ref[:, H:] = y[HALF:]


def _project(embt, W):
    return pl.pallas_call(
        _proj_body,
        grid=(NBLK,),
        in_specs=[
            pl.BlockSpec((D, ROW_BLK), lambda i: (0, i)),
            pl.BlockSpec((H, D), lambda i: (0, 0)),
        ],
        out_specs=pl.BlockSpec((HALF, 2 * H), lambda i: (i, 0)),
        out_shape=jax.ShapeDtypeStruct((VP // 2, 2 * H), jnp.float32),
    )(embt, W)


_mesh = plsc.VectorSubcoreMesh(core_axis_name="c", subcore_axis_name="s")


NBUF = 8
DEPTH = NBUF // 2
NLAP = NCHUNK // NBUF
SLAB = 8
BHALF = B // 2


def _sig16(v):
    return ((v >> SH) << SH) + ((v & (HALF - 1)) << 1) + ((v >> (SH - 1)) & 1)


@functools.partial(
    pl.kernel,
    mesh=_mesh,
    compiler_params=pltpu.CompilerParams(
        use_tc_tiling_on_sc=False, needs_layout_passes=False),
    out_type=jax.ShapeDtypeStruct((N, H), jnp.float32),
    scratch_types=[
        pltpu.VMEM((SLAB * B,), jnp.int32),
        pltpu.VMEM((NBUF, CHUNK), jnp.int32),
        pltpu.VMEM((NBUF, CHUNK, H), jnp.float32),
        [pltpu.SemaphoreType.DMA] * NBUF,
        [pltpu.SemaphoreType.DMA] * NBUF,
    ],
)
def _gather(proj_hbm, sentt_hbm, out_hbm, slab_v, cidx_v, rows_v, gsem, ssem):
    wid = lax.axis_index("s") * NC + lax.axis_index("c")
    base = wid * ROWS_PER_W
    l_lo = jnp.minimum(base // B, L - SLAB)

    for r in range(SLAB):
        pltpu.sync_copy(sentt_hbm.at[l_lo + r],
                        slab_v.at[pl.ds(r * B, B)])

    iota = jax.lax.iota(jnp.int32, 16)

    def build(j, b):
        m0 = base + j * CHUNK
        l = m0 // B
        col0 = (m0 - l * B) >> 1
        flat0 = (l - l_lo) * B + col0
        dst = cidx_v.at[b]
        for t in range(4):
            ev = slab_v[pl.ds(flat0 + 16 * t, 16)]
            od = slab_v[pl.ds(flat0 + BHALF + 16 * t, 16)]
            plsc.store_scatter(dst, [iota * 2 + 32 * t], _sig16(ev))
            plsc.store_scatter(dst, [iota * 2 + 32 * t + 1], _sig16(od))

    def g_copy(b):
        return pltpu.make_async_copy(
            proj_hbm.at[cidx_v.at[b]], rows_v.at[b], gsem[b])

    def s_copy(j, b):
        return pltpu.make_async_copy(
            rows_v.at[b], out_hbm.at[pl.ds(base + j * CHUNK, CHUNK)],
            ssem[b])

    def emit(j, b, issue_next, wait_next_store):
        g_copy(b).wait()
        s_copy(j, b).start()
        if issue_next:
            nxt = j + DEPTH
            b2 = (b + DEPTH) % NBUF
            if wait_next_store:
                s_copy(nxt - NBUF, b2).wait()
            build(nxt, b2)
            g_copy(b2).start()

    for b in range(DEPTH):
        build(b, b)
        g_copy(b).start()
    for b in range(NBUF):
        emit(b, b, True, b >= DEPTH)

    def lap(t, carry):
        j0 = t * NBUF
        for b in range(NBUF):
            emit(j0 + b, b, True, True)
        return carry

    lax.fori_loop(1, NLAP - 1, lap, 0)

    j0 = (NLAP - 1) * NBUF
    for b in range(NBUF):
        emit(j0 + b, b, b < DEPTH, True)
    for b in range(NBUF):
        s_copy(j0 + b, b).wait()


LSTEP = 8


def _trans_body(i_ref, g_ref, out_ref):
    dn = (((0,), (1,)), ((), ()))
    for s in range(LSTEP):
        blk = g_ref[pl.ds(s * BHALF, BHALF), :]
        out_ref[s, :, :BHALF] = lax.dot_general(
            i_ref[...], blk[:, :H], dn, preferred_element_type=jnp.float32)
        out_ref[s, :, BHALF:] = lax.dot_general(
            i_ref[...], blk[:, H:], dn, preferred_element_type=jnp.float32)


def _transpose_out(g2, eye):
    return pl.pallas_call(
        _trans_body,
        grid=(L // LSTEP,),
        in_specs=[
            pl.BlockSpec((H, H), lambda i: (0, 0)),
            pl.BlockSpec((LSTEP * BHALF, 2 * H), lambda i: (i, 0)),
        ],
        out_specs=pl.BlockSpec((LSTEP, H, B), lambda i: (i, 0, 0)),
        out_shape=jax.ShapeDtypeStruct((L, H, B), jnp.float32),
    )(eye, g2)


def kernel(sent1, sent2, emb, W):
    proj = _project(emb.T, W).reshape(VP, H)
    eye = jnp.eye(H, dtype=jnp.float32)
    g1 = _gather(proj, sent1.T.astype(jnp.int32))
    g2 = _gather(proj, sent2.T.astype(jnp.int32))
    o1 = _transpose_out(g1.reshape(N // 2, 2 * H), eye)
    o2 = _transpose_out(g2.reshape(N // 2, 2 * H), eye)
    return (o1.transpose(2, 0, 1), o2.transpose(2, 0, 1))
